# baseline (speedup 1.0000x reference)
import functools

import jax
import jax.numpy as jnp
from jax import lax
from jax.experimental import pallas as pl
from jax.experimental.pallas import tpu as pltpu
from jax.experimental.pallas import tpu_sc as plsc

_E = 64
_K = 8
_NG = 8
_KG = 4
_GS = _E // _NG
_D = 1024
_F = 512
_T = 128
_SCALE = 2.5
_EPB = 2
_L = 16
_NCHUNK = _T // _L


def _logits_kernel(x_ref, gw_ref, gb_ref, scores_ref, s4c_ref):
    lt = lax.dot_general(gw_ref[...], x_ref[...], (((1,), (1,)), ((), ())),
                         preferred_element_type=jnp.float32)
    sc = jax.nn.sigmoid(lt)
    scores_ref[0] = sc
    s4c_ref[0] = sc + gb_ref[...]


def _sc_gate_body(scores_hbm, s4c_hbm, wd_hbm, scores_v, s4_v, m4_v, out_v):
    wid = lax.axis_index("s") * 2 + lax.axis_index("c")

    @pl.when(wid < _NCHUNK)
    def _():
        pltpu.sync_copy(scores_hbm.at[wid], scores_v)
        pltpu.sync_copy(s4c_hbm.at[wid], s4_v)

        neg = jnp.full((_L,), -jnp.inf, jnp.float32)

        gs = []
        for g in range(_NG):
            m1 = neg
            m2 = neg
            for j in range(_GS):
                v = s4_v[g * _GS + j]
                gt = v > m1
                m2 = jnp.where(gt, m1, jnp.maximum(m2, v))
                m1 = jnp.maximum(m1, v)
            gs.append(m1 + m2)

        gsel = [jnp.zeros((_L,), jnp.bool_) for _ in range(_NG)]
        for _it in range(_KG):
            best = neg
            besti = jnp.full((_L,), _NG, jnp.int32)
            for g in range(_NG):
                v = jnp.where(gsel[g], neg, gs[g])
                take = v > best
                besti = jnp.where(take, g, besti)
                best = jnp.where(take, v, best)
            for g in range(_NG):
                gsel[g] = jnp.logical_or(gsel[g], besti == g)

        for e in range(_E):
            m4_v[e] = jnp.where(gsel[e // _GS], s4_v[e], neg)
            out_v[e] = jnp.zeros((_L,), jnp.float32)

        denom = jnp.zeros((_L,), jnp.float32)
        for _it in range(_K):
            def _scan(e2, best):
                return jnp.maximum(best, m4_v[e2])
            best = lax.fori_loop(0, _E, _scan, neg, unroll=8)

            def _mark(e2, carry):
                done, sc_best = carry
                v = m4_v[e2]
                pick = jnp.logical_and(v == best, done == 0.0)
                m4_v[e2] = jnp.where(pick, neg, v)
                sc_best = jnp.where(pick, scores_v[e2], sc_best)
                out_v[e2] = jnp.where(pick, scores_v[e2], out_v[e2])
                return jnp.where(pick, 1.0, done), sc_best
            _, sc_win = lax.fori_loop(
                0, _E, _mark,
                (jnp.zeros((_L,), jnp.float32), jnp.zeros((_L,), jnp.float32)),
                unroll=8)
            denom = denom + sc_win

        scale = _SCALE / (denom + 1e-20)

        def _norm(e2, _):
            out_v[e2] = out_v[e2] * scale
            return 0
        lax.fori_loop(0, _E, _norm, 0, unroll=8)

        pltpu.sync_copy(out_v, wd_hbm.at[wid])


_sc_gate = functools.partial(
    pl.kernel,
    out_type=jax.ShapeDtypeStruct((_NCHUNK, _E, _L), jnp.float32),
    mesh=plsc.VectorSubcoreMesh(core_axis_name="c", subcore_axis_name="s"),
    scratch_types=[
        pltpu.VMEM((_E, _L), jnp.float32),
        pltpu.VMEM((_E, _L), jnp.float32),
        pltpu.VMEM((_E, _L), jnp.float32),
        pltpu.VMEM((_E, _L), jnp.float32),
    ],
)(_sc_gate_body)


def _moe_kernel(x_ref, wd_ref, wg_ref, wu_ref, wd_w_ref, out_ref):
    e = pl.program_id(0)
    x = x_ref[...]
    lane = lax.broadcasted_iota(jnp.int32, (_T, _E), 1)
    contrib = jnp.zeros((_T, _D), jnp.float32)
    for j in range(_EPB):
        h = jnp.dot(x, wg_ref[j], preferred_element_type=jnp.float32)
        u = jnp.dot(x, wu_ref[j], preferred_element_type=jnp.float32)
        act = (h * jax.nn.sigmoid(h)) * u
        y = jnp.dot(act, wd_w_ref[j], preferred_element_type=jnp.float32)
        wcol = jnp.sum(jnp.where(lane == _EPB * e + j, wd_ref[...], 0.0),
                       axis=1, keepdims=True)
        contrib = contrib + y * wcol

    @pl.when(e == 0)
    def _():
        out_ref[...] = contrib

    @pl.when(e > 0)
    def _():
        out_ref[...] += contrib


def kernel(tt_input, gate_w, gate_bias, w_gate, w_up, w_down):
    gb_col = gate_bias.reshape(_E, 1)
    scores_c, s4c_c = pl.pallas_call(
        _logits_kernel,
        grid=(_NCHUNK,),
        in_specs=[
            pl.BlockSpec((_L, _D), lambda c: (c, 0)),
            pl.BlockSpec((_E, _D), lambda c: (0, 0)),
            pl.BlockSpec((_E, 1), lambda c: (0, 0)),
        ],
        out_specs=[
            pl.BlockSpec((1, _E, _L), lambda c: (c, 0, 0)),
            pl.BlockSpec((1, _E, _L), lambda c: (c, 0, 0)),
        ],
        out_shape=[
            jax.ShapeDtypeStruct((_NCHUNK, _E, _L), jnp.float32),
            jax.ShapeDtypeStruct((_NCHUNK, _E, _L), jnp.float32),
        ],
    )(tt_input, gate_w, gb_col)

    wd3 = _sc_gate(scores_c, s4c_c)
    wd = jnp.transpose(wd3, (0, 2, 1)).reshape(_T, _E)

    return pl.pallas_call(
        _moe_kernel,
        grid=(_E // _EPB,),
        in_specs=[
            pl.BlockSpec((_T, _D), lambda e: (0, 0)),
            pl.BlockSpec((_T, _E), lambda e: (0, 0)),
            pl.BlockSpec((_EPB, _D, _F), lambda e: (e, 0, 0)),
            pl.BlockSpec((_EPB, _D, _F), lambda e: (e, 0, 0)),
            pl.BlockSpec((_EPB, _F, _D), lambda e: (e, 0, 0)),
        ],
        out_specs=pl.BlockSpec((_T, _D), lambda e: (0, 0)),
        out_shape=jax.ShapeDtypeStruct((_T, _D), jnp.float32),
        compiler_params=pltpu.CompilerParams(
            dimension_semantics=("arbitrary",),
        ),
    )(tt_input, wd, w_gate, w_up, w_down)


# speedup vs baseline: 1.0349x; 1.0349x over previous
import functools

import jax
import jax.numpy as jnp
from jax import lax
from jax.experimental import pallas as pl
from jax.experimental.pallas import tpu as pltpu
from jax.experimental.pallas import tpu_sc as plsc

_E = 64
_K = 8
_NG = 8
_KG = 4
_GS = _E // _NG
_D = 1024
_F = 512
_T = 128
_SCALE = 2.5
_EPB = 2
_L = 16
_NCHUNK = _T // _L


def _logits_kernel(x_ref, gw_ref, gb_ref, scores_ref, s4c_ref):
    lt = lax.dot_general(gw_ref[...], x_ref[...], (((1,), (1,)), ((), ())),
                         preferred_element_type=jnp.float32)
    sc = jax.nn.sigmoid(lt)
    scores_ref[0] = sc
    s4c_ref[0] = sc + gb_ref[...]


def _sc_gate_body(scores_hbm, s4c_hbm, wd_hbm, scores_v, s4_v, m4_v, out_v):
    wid = lax.axis_index("s") * 2 + lax.axis_index("c")

    @pl.when(wid < _NCHUNK)
    def _():
        pltpu.sync_copy(scores_hbm.at[wid], scores_v)
        pltpu.sync_copy(s4c_hbm.at[wid], s4_v)

        neg = jnp.full((_L,), -jnp.inf, jnp.float32)
        one = jnp.full((_L,), 1.0, jnp.float32)

        gs = []
        for g in range(_NG):
            m1 = neg
            m2 = neg
            for j in range(_GS):
                v = s4_v[g * _GS + j]
                gt = v > m1
                m2 = jnp.where(gt, m1, jnp.maximum(m2, v))
                m1 = jnp.maximum(m1, v)
            gs.append(m1 + m2)

        gsel = [jnp.zeros((_L,), jnp.bool_) for _ in range(_NG)]
        for _it in range(_KG):
            best = neg
            besti = jnp.full((_L,), _NG, jnp.int32)
            for g in range(_NG):
                v = jnp.where(gsel[g], neg, gs[g])
                take = v > best
                besti = jnp.where(take, g, besti)
                best = jnp.where(take, v, best)
            for g in range(_NG):
                gsel[g] = jnp.logical_or(gsel[g], besti == g)

        for e in range(_E):
            m4_v[e] = jnp.where(gsel[e // _GS], s4_v[e], neg)

        for _it in range(_K):
            def _scan(e2, best):
                return jnp.maximum(best, m4_v[e2])
            best = lax.fori_loop(0, _E, _scan, neg, unroll=8)

            def _mark(e2, done):
                v = m4_v[e2]
                pick = jnp.logical_and(v == best, done == 0.0)
                m4_v[e2] = jnp.where(pick, neg, v)
                return jnp.where(pick, one, done)
            lax.fori_loop(0, _E, _mark, jnp.zeros((_L,), jnp.float32),
                          unroll=8)

        denom = jnp.zeros((_L,), jnp.float32)
        for e in range(_E):
            w = jnp.where(
                jnp.logical_and(gsel[e // _GS], m4_v[e] == neg),
                scores_v[e], 0.0)
            out_v[e, 0] = w
            denom = denom + w

        scale = _SCALE / (denom + 1e-20)
        for e in range(_E):
            out_v[e, 0] = out_v[e, 0] * scale

        pltpu.sync_copy(out_v, wd_hbm.at[:, wid])


_sc_gate = functools.partial(
    pl.kernel,
    out_type=jax.ShapeDtypeStruct((_E, _NCHUNK, 1, _L), jnp.float32),
    mesh=plsc.VectorSubcoreMesh(core_axis_name="c", subcore_axis_name="s"),
    scratch_types=[
        pltpu.VMEM((_E, _L), jnp.float32),
        pltpu.VMEM((_E, _L), jnp.float32),
        pltpu.VMEM((_E, _L), jnp.float32),
        pltpu.VMEM((_E, 1, _L), jnp.float32),
    ],
)(_sc_gate_body)


def _moe_kernel(x_ref, wdT_ref, wg_ref, wu_ref, wd_w_ref, out_ref, wd_scr):
    e = pl.program_id(0)

    @pl.when(e == 0)
    def _():
        ii = lax.broadcasted_iota(jnp.int32, (_E, _E), 0)
        jj = lax.broadcasted_iota(jnp.int32, (_E, _E), 1)
        eye = (ii == jj).astype(jnp.float32)
        wd_scr[...] = lax.dot_general(wdT_ref[...], eye,
                                      (((0,), (0,)), ((), ())),
                                      preferred_element_type=jnp.float32)

    x = x_ref[...]
    lane = lax.broadcasted_iota(jnp.int32, (_T, _E), 1)
    contrib = jnp.zeros((_T, _D), jnp.float32)
    for j in range(_EPB):
        h = jnp.dot(x, wg_ref[j], preferred_element_type=jnp.float32)
        u = jnp.dot(x, wu_ref[j], preferred_element_type=jnp.float32)
        act = (h * jax.nn.sigmoid(h)) * u
        y = jnp.dot(act, wd_w_ref[j], preferred_element_type=jnp.float32)
        wcol = jnp.sum(jnp.where(lane == _EPB * e + j, wd_scr[...], 0.0),
                       axis=1, keepdims=True)
        contrib = contrib + y * wcol

    @pl.when(e == 0)
    def _():
        out_ref[...] = contrib

    @pl.when(e > 0)
    def _():
        out_ref[...] += contrib


def kernel(tt_input, gate_w, gate_bias, w_gate, w_up, w_down):
    gb_col = gate_bias.reshape(_E, 1)
    scores_c, s4c_c = pl.pallas_call(
        _logits_kernel,
        grid=(_NCHUNK,),
        in_specs=[
            pl.BlockSpec((_L, _D), lambda c: (c, 0)),
            pl.BlockSpec((_E, _D), lambda c: (0, 0)),
            pl.BlockSpec((_E, 1), lambda c: (0, 0)),
        ],
        out_specs=[
            pl.BlockSpec((1, _E, _L), lambda c: (c, 0, 0)),
            pl.BlockSpec((1, _E, _L), lambda c: (c, 0, 0)),
        ],
        out_shape=[
            jax.ShapeDtypeStruct((_NCHUNK, _E, _L), jnp.float32),
            jax.ShapeDtypeStruct((_NCHUNK, _E, _L), jnp.float32),
        ],
    )(tt_input, gate_w, gb_col)

    wdT = _sc_gate(scores_c, s4c_c).reshape(_E, _T)

    return pl.pallas_call(
        _moe_kernel,
        grid=(_E // _EPB,),
        in_specs=[
            pl.BlockSpec((_T, _D), lambda e: (0, 0)),
            pl.BlockSpec((_E, _T), lambda e: (0, 0)),
            pl.BlockSpec((_EPB, _D, _F), lambda e: (e, 0, 0)),
            pl.BlockSpec((_EPB, _D, _F), lambda e: (e, 0, 0)),
            pl.BlockSpec((_EPB, _F, _D), lambda e: (e, 0, 0)),
        ],
        out_specs=pl.BlockSpec((_T, _D), lambda e: (0, 0)),
        out_shape=jax.ShapeDtypeStruct((_T, _D), jnp.float32),
        scratch_shapes=[pltpu.VMEM((_T, _E), jnp.float32)],
        compiler_params=pltpu.CompilerParams(
            dimension_semantics=("arbitrary",),
        ),
    )(tt_input, wdT, w_gate, w_up, w_down)


# speedup vs baseline: 1.0450x; 1.0097x over previous
import functools

import jax
import jax.numpy as jnp
from jax import lax
from jax.experimental import pallas as pl
from jax.experimental.pallas import tpu as pltpu
from jax.experimental.pallas import tpu_sc as plsc

_E = 64
_K = 8
_NG = 8
_KG = 4
_GS = _E // _NG
_D = 1024
_F = 512
_T = 128
_SCALE = 2.5
_EPB = 2
_L = 16
_NCHUNK = _T // _L


def _logits_kernel(x_ref, gw_ref, gb_ref, scores_ref, s4c_ref):
    lt = lax.dot_general(gw_ref[...], x_ref[...], (((1,), (1,)), ((), ())),
                         preferred_element_type=jnp.float32)
    sc = jax.nn.sigmoid(lt)
    s4c = sc + gb_ref[...]
    for c in range(_NCHUNK):
        scores_ref[c] = sc[:, c * _L:(c + 1) * _L]
        s4c_ref[c] = s4c[:, c * _L:(c + 1) * _L]


def _sc_gate_body(scores_hbm, s4c_hbm, wd_hbm, scores_v, s4_v, m4_v, out_v):
    wid = lax.axis_index("s") * 2 + lax.axis_index("c")

    @pl.when(wid < _NCHUNK)
    def _():
        pltpu.sync_copy(scores_hbm.at[wid], scores_v)
        pltpu.sync_copy(s4c_hbm.at[wid], s4_v)

        neg = jnp.full((_L,), -jnp.inf, jnp.float32)
        one = jnp.full((_L,), 1.0, jnp.float32)

        gs = []
        for g in range(_NG):
            m1 = neg
            m2 = neg
            for j in range(_GS):
                v = s4_v[g * _GS + j]
                gt = v > m1
                m2 = jnp.where(gt, m1, jnp.maximum(m2, v))
                m1 = jnp.maximum(m1, v)
            gs.append(m1 + m2)

        gsel = [jnp.zeros((_L,), jnp.bool_) for _ in range(_NG)]
        for _it in range(_KG):
            best = neg
            besti = jnp.full((_L,), _NG, jnp.int32)
            for g in range(_NG):
                v = jnp.where(gsel[g], neg, gs[g])
                take = v > best
                besti = jnp.where(take, g, besti)
                best = jnp.where(take, v, best)
            for g in range(_NG):
                gsel[g] = jnp.logical_or(gsel[g], besti == g)

        for e in range(_E):
            m4_v[e] = jnp.where(gsel[e // _GS], s4_v[e], neg)

        for _it in range(_K):
            def _scan(e2, best):
                return jnp.maximum(best, m4_v[e2])
            best = lax.fori_loop(0, _E, _scan, neg, unroll=16)

            def _mark(e2, done):
                v = m4_v[e2]
                pick = jnp.logical_and(v == best, done == 0.0)
                m4_v[e2] = jnp.where(pick, neg, v)
                return jnp.where(pick, one, done)
            lax.fori_loop(0, _E, _mark, jnp.zeros((_L,), jnp.float32),
                          unroll=16)

        denom = jnp.zeros((_L,), jnp.float32)
        for e in range(_E):
            w = jnp.where(
                jnp.logical_and(gsel[e // _GS], m4_v[e] == neg),
                scores_v[e], 0.0)
            out_v[e, 0] = w
            denom = denom + w

        scale = _SCALE / (denom + 1e-20)
        for e in range(_E):
            out_v[e, 0] = out_v[e, 0] * scale

        pltpu.sync_copy(out_v, wd_hbm.at[:, wid])


_sc_gate = functools.partial(
    pl.kernel,
    out_type=jax.ShapeDtypeStruct((_E, _NCHUNK, 1, _L), jnp.float32),
    mesh=plsc.VectorSubcoreMesh(core_axis_name="c", subcore_axis_name="s"),
    scratch_types=[
        pltpu.VMEM((_E, _L), jnp.float32),
        pltpu.VMEM((_E, _L), jnp.float32),
        pltpu.VMEM((_E, _L), jnp.float32),
        pltpu.VMEM((_E, 1, _L), jnp.float32),
    ],
)(_sc_gate_body)


def _moe_kernel(x_ref, wdT_ref, wg_ref, wu_ref, wd_w_ref, out_ref, wd_scr):
    e = pl.program_id(0)

    @pl.when(e == 0)
    def _():
        ii = lax.broadcasted_iota(jnp.int32, (_E, _E), 0)
        jj = lax.broadcasted_iota(jnp.int32, (_E, _E), 1)
        eye = (ii == jj).astype(jnp.float32)
        wd_scr[...] = lax.dot_general(wdT_ref[...], eye,
                                      (((0,), (0,)), ((), ())),
                                      preferred_element_type=jnp.float32)

    x = x_ref[...]
    lane = lax.broadcasted_iota(jnp.int32, (_T, _E), 1)
    contrib = jnp.zeros((_T, _D), jnp.float32)
    for j in range(_EPB):
        h = jnp.dot(x, wg_ref[j], preferred_element_type=jnp.float32)
        u = jnp.dot(x, wu_ref[j], preferred_element_type=jnp.float32)
        act = (h * jax.nn.sigmoid(h)) * u
        y = jnp.dot(act, wd_w_ref[j], preferred_element_type=jnp.float32)
        wcol = jnp.sum(jnp.where(lane == _EPB * e + j, wd_scr[...], 0.0),
                       axis=1, keepdims=True)
        contrib = contrib + y * wcol

    @pl.when(e == 0)
    def _():
        out_ref[...] = contrib

    @pl.when(e > 0)
    def _():
        out_ref[...] += contrib


def kernel(tt_input, gate_w, gate_bias, w_gate, w_up, w_down):
    gb_col = gate_bias.reshape(_E, 1)
    scores_c, s4c_c = pl.pallas_call(
        _logits_kernel,
        in_specs=[
            pl.BlockSpec((_T, _D), lambda: (0, 0)),
            pl.BlockSpec((_E, _D), lambda: (0, 0)),
            pl.BlockSpec((_E, 1), lambda: (0, 0)),
        ],
        out_specs=[
            pl.BlockSpec((_NCHUNK, _E, _L), lambda: (0, 0, 0)),
            pl.BlockSpec((_NCHUNK, _E, _L), lambda: (0, 0, 0)),
        ],
        out_shape=[
            jax.ShapeDtypeStruct((_NCHUNK, _E, _L), jnp.float32),
            jax.ShapeDtypeStruct((_NCHUNK, _E, _L), jnp.float32),
        ],
    )(tt_input, gate_w, gb_col)

    wdT = _sc_gate(scores_c, s4c_c).reshape(_E, _T)

    return pl.pallas_call(
        _moe_kernel,
        grid=(_E // _EPB,),
        in_specs=[
            pl.BlockSpec((_T, _D), lambda e: (0, 0)),
            pl.BlockSpec((_E, _T), lambda e: (0, 0)),
            pl.BlockSpec((_EPB, _D, _F), lambda e: (e, 0, 0)),
            pl.BlockSpec((_EPB, _D, _F), lambda e: (e, 0, 0)),
            pl.BlockSpec((_EPB, _F, _D), lambda e: (e, 0, 0)),
        ],
        out_specs=pl.BlockSpec((_T, _D), lambda e: (0, 0)),
        out_shape=jax.ShapeDtypeStruct((_T, _D), jnp.float32),
        scratch_shapes=[pltpu.VMEM((_T, _E), jnp.float32)],
        compiler_params=pltpu.CompilerParams(
            dimension_semantics=("arbitrary",),
        ),
    )(tt_input, wdT, w_gate, w_up, w_down)


# speedup vs baseline: 1.0707x; 1.0246x over previous
import functools

import jax
import jax.numpy as jnp
from jax import lax
from jax.experimental import pallas as pl
from jax.experimental.pallas import tpu as pltpu
from jax.experimental.pallas import tpu_sc as plsc

_E = 64
_K = 8
_NG = 8
_KG = 4
_GS = _E // _NG
_D = 1024
_F = 512
_T = 128
_SCALE = 2.5
_EPB = 2
_L = 16
_NCHUNK = _T // _L


def _logits_kernel(x_ref, gw_ref, gb_ref, scores_ref, s4c_ref):
    lt = lax.dot_general(gw_ref[...], x_ref[...], (((1,), (1,)), ((), ())),
                         preferred_element_type=jnp.float32)
    sc = jax.nn.sigmoid(lt)
    s4c = sc + gb_ref[...]
    for c in range(_NCHUNK):
        scores_ref[c] = sc[:, c * _L:(c + 1) * _L]
        s4c_ref[c] = s4c[:, c * _L:(c + 1) * _L]


def _sc_gate_body(scores_hbm, s4c_hbm, wd_hbm, scores_v, s4_v, m4_v, out_v):
    wid = lax.axis_index("s") * 2 + lax.axis_index("c")

    @pl.when(wid < _NCHUNK)
    def _():
        pltpu.sync_copy(scores_hbm.at[wid], scores_v)
        pltpu.sync_copy(s4c_hbm.at[wid], s4_v)

        neg = jnp.full((_L,), -jnp.inf, jnp.float32)
        one = jnp.full((_L,), 1.0, jnp.float32)

        gs = []
        for g in range(_NG):
            m1 = neg
            m2 = neg
            for j in range(_GS):
                v = s4_v[g * _GS + j]
                gt = v > m1
                m2 = jnp.where(gt, m1, jnp.maximum(m2, v))
                m1 = jnp.maximum(m1, v)
            gs.append(m1 + m2)

        gsel = [jnp.zeros((_L,), jnp.bool_) for _ in range(_NG)]
        for _it in range(_KG):
            best = neg
            besti = jnp.full((_L,), _NG, jnp.int32)
            for g in range(_NG):
                v = jnp.where(gsel[g], neg, gs[g])
                take = v > best
                besti = jnp.where(take, g, besti)
                best = jnp.where(take, v, best)
            for g in range(_NG):
                gsel[g] = jnp.logical_or(gsel[g], besti == g)

        for e in range(_E):
            m4_v[e] = jnp.where(gsel[e // _GS], s4_v[e], neg)

        for _it in range(_K):
            def _scan(e2, best):
                return jnp.maximum(best, m4_v[e2])
            best = lax.fori_loop(0, _E, _scan, neg, unroll=16)

            def _mark(e2, done):
                v = m4_v[e2]
                pick = jnp.logical_and(v == best, done == 0.0)
                m4_v[e2] = jnp.where(pick, neg, v)
                return jnp.where(pick, one, done)
            lax.fori_loop(0, _E, _mark, jnp.zeros((_L,), jnp.float32),
                          unroll=16)

        denom = jnp.zeros((_L,), jnp.float32)
        for e in range(_E):
            w = jnp.where(
                jnp.logical_and(gsel[e // _GS], m4_v[e] == neg),
                scores_v[e], 0.0)
            out_v[e, 0] = w
            denom = denom + w

        scale = _SCALE / (denom + 1e-20)
        for e in range(_E):
            out_v[e, 0] = out_v[e, 0] * scale

        pltpu.sync_copy(out_v, wd_hbm.at[:, wid])


_sc_gate = functools.partial(
    pl.kernel,
    out_type=jax.ShapeDtypeStruct((_E, _NCHUNK, 1, _L), jnp.float32),
    mesh=plsc.VectorSubcoreMesh(core_axis_name="c", subcore_axis_name="s"),
    scratch_types=[
        pltpu.VMEM((_E, _L), jnp.float32),
        pltpu.VMEM((_E, _L), jnp.float32),
        pltpu.VMEM((_E, _L), jnp.float32),
        pltpu.VMEM((_E, 1, _L), jnp.float32),
    ],
)(_sc_gate_body)


_EA = 8


def _mlp(x, wg, wu, wd):
    h = jnp.dot(x, wg, preferred_element_type=jnp.float32)
    u = jnp.dot(x, wu, preferred_element_type=jnp.float32)
    act = (h * jax.nn.sigmoid(h)) * u
    return jnp.dot(act, wd, preferred_element_type=jnp.float32)


def _moe_a_kernel(x_ref, wg_ref, wu_ref, wd_w_ref, y_ref):
    x = x_ref[...]
    for j in range(_EPB):
        y_ref[j] = _mlp(x, wg_ref[j], wu_ref[j], wd_w_ref[j]).astype(
            jnp.bfloat16)


def _moe_b_kernel(x_ref, wdT_ref, y8_ref, wg_ref, wu_ref, wd_w_ref, out_ref,
                  wd_scr):
    e = pl.program_id(0)
    lane = lax.broadcasted_iota(jnp.int32, (_T, _E), 1)

    def _wcol(idx):
        return jnp.sum(jnp.where(lane == idx, wd_scr[...], 0.0),
                       axis=1, keepdims=True)

    @pl.when(e == 0)
    def _():
        ii = lax.broadcasted_iota(jnp.int32, (_E, _E), 0)
        jj = lax.broadcasted_iota(jnp.int32, (_E, _E), 1)
        eye = (ii == jj).astype(jnp.float32)
        wd_scr[...] = lax.dot_general(wdT_ref[...], eye,
                                      (((0,), (0,)), ((), ())),
                                      preferred_element_type=jnp.float32)
        out_ref[...] = jnp.zeros((_T, _D), jnp.float32)

    @pl.when(e < _EA // _EPB)
    def _():
        contrib_a = jnp.zeros((_T, _D), jnp.float32)
        for j in range(_EPB):
            contrib_a = contrib_a + (
                y8_ref[j].astype(jnp.float32) * _wcol(_EPB * e + j))
        out_ref[...] += contrib_a

    x = x_ref[...]
    contrib = jnp.zeros((_T, _D), jnp.float32)
    for j in range(_EPB):
        y = _mlp(x, wg_ref[j], wu_ref[j], wd_w_ref[j])
        contrib = contrib + y * _wcol(_EA + _EPB * e + j)
    out_ref[...] += contrib


def kernel(tt_input, gate_w, gate_bias, w_gate, w_up, w_down):
    gb_col = gate_bias.reshape(_E, 1)
    scores_c, s4c_c = pl.pallas_call(
        _logits_kernel,
        in_specs=[
            pl.BlockSpec((_T, _D), lambda: (0, 0)),
            pl.BlockSpec((_E, _D), lambda: (0, 0)),
            pl.BlockSpec((_E, 1), lambda: (0, 0)),
        ],
        out_specs=[
            pl.BlockSpec((_NCHUNK, _E, _L), lambda: (0, 0, 0)),
            pl.BlockSpec((_NCHUNK, _E, _L), lambda: (0, 0, 0)),
        ],
        out_shape=[
            jax.ShapeDtypeStruct((_NCHUNK, _E, _L), jnp.float32),
            jax.ShapeDtypeStruct((_NCHUNK, _E, _L), jnp.float32),
        ],
    )(tt_input, gate_w, gb_col)

    wdT = _sc_gate(scores_c, s4c_c).reshape(_E, _T)

    y8 = pl.pallas_call(
        _moe_a_kernel,
        grid=(_EA // _EPB,),
        in_specs=[
            pl.BlockSpec((_T, _D), lambda e: (0, 0)),
            pl.BlockSpec((_EPB, _D, _F), lambda e: (e, 0, 0)),
            pl.BlockSpec((_EPB, _D, _F), lambda e: (e, 0, 0)),
            pl.BlockSpec((_EPB, _F, _D), lambda e: (e, 0, 0)),
        ],
        out_specs=pl.BlockSpec((_EPB, _T, _D), lambda e: (e, 0, 0)),
        out_shape=jax.ShapeDtypeStruct((_EA, _T, _D), jnp.bfloat16),
        compiler_params=pltpu.CompilerParams(
            dimension_semantics=("arbitrary",),
        ),
    )(tt_input, w_gate, w_up, w_down)

    nb = (_E - _EA) // _EPB
    return pl.pallas_call(
        _moe_b_kernel,
        grid=(nb,),
        in_specs=[
            pl.BlockSpec((_T, _D), lambda e: (0, 0)),
            pl.BlockSpec((_E, _T), lambda e: (0, 0)),
            pl.BlockSpec((_EPB, _T, _D),
                         lambda e: (jnp.minimum(e, _EA // _EPB - 1), 0, 0)),
            pl.BlockSpec((_EPB, _D, _F), lambda e: (e + _EA // _EPB, 0, 0)),
            pl.BlockSpec((_EPB, _D, _F), lambda e: (e + _EA // _EPB, 0, 0)),
            pl.BlockSpec((_EPB, _F, _D), lambda e: (e + _EA // _EPB, 0, 0)),
        ],
        out_specs=pl.BlockSpec((_T, _D), lambda e: (0, 0)),
        out_shape=jax.ShapeDtypeStruct((_T, _D), jnp.float32),
        scratch_shapes=[pltpu.VMEM((_T, _E), jnp.float32)],
        compiler_params=pltpu.CompilerParams(
            dimension_semantics=("arbitrary",),
        ),
    )(tt_input, wdT, y8, w_gate, w_up, w_down)


# speedup vs baseline: 1.0718x; 1.0010x over previous
import functools

import jax
import jax.numpy as jnp
from jax import lax
from jax.experimental import pallas as pl
from jax.experimental.pallas import tpu as pltpu
from jax.experimental.pallas import tpu_sc as plsc

_E = 64
_K = 8
_NG = 8
_KG = 4
_GS = _E // _NG
_D = 1024
_F = 512
_T = 128
_SCALE = 2.5
_EPB = 2
_L = 16
_NCHUNK = _T // _L


def _logits_kernel(x_ref, gw_ref, gb_ref, scores_ref, s4c_ref):
    lt = lax.dot_general(gw_ref[...], x_ref[...], (((1,), (1,)), ((), ())),
                         preferred_element_type=jnp.float32)
    sc = jax.nn.sigmoid(lt)
    s4c = sc + gb_ref[...]
    for c in range(_NCHUNK):
        scores_ref[c] = sc[:, c * _L:(c + 1) * _L]
        s4c_ref[c] = s4c[:, c * _L:(c + 1) * _L]


def _sc_gate_body(scores_hbm, s4c_hbm, wd_hbm, scores_v, s4_v, m4_v, out_v,
                  gs_v, gact_v):
    wid = lax.axis_index("s") * 2 + lax.axis_index("c")

    @pl.when(wid < _NCHUNK)
    def _():
        pltpu.sync_copy(scores_hbm.at[wid], scores_v)
        pltpu.sync_copy(s4c_hbm.at[wid], s4_v)

        neg = jnp.full((_L,), -jnp.inf, jnp.float32)
        one = jnp.full((_L,), 1.0, jnp.float32)

        def _grp(g, _):
            def _run(j, c):
                m1, m2 = c
                v = s4_v[g * _GS + j]
                gt = v > m1
                return (jnp.maximum(m1, v),
                        jnp.where(gt, m1, jnp.maximum(m2, v)))
            m1, m2 = lax.fori_loop(0, _GS, _run, (neg, neg))
            gs_v[g] = m1 + m2
            gact_v[g] = jnp.zeros((_L,), jnp.float32)
            return 0
        lax.fori_loop(0, _NG, _grp, 0)

        def _g4(_it, _):
            def _gscan(g, best):
                v = jnp.where(gact_v[g] > 0, neg, gs_v[g])
                return jnp.maximum(best, v)
            best = lax.fori_loop(0, _NG, _gscan, neg)

            def _gmark(g, done):
                v = jnp.where(gact_v[g] > 0, neg, gs_v[g])
                pick = jnp.logical_and(v == best, done == 0.0)
                gact_v[g] = jnp.where(pick, one, gact_v[g])
                return jnp.where(pick, one, done)
            lax.fori_loop(0, _NG, _gmark, jnp.zeros((_L,), jnp.float32))
            return 0
        lax.fori_loop(0, _KG, _g4, 0)

        def _init(e2, _):
            m4_v[e2] = jnp.where(gact_v[e2 // _GS] > 0, s4_v[e2], neg)
            return 0
        lax.fori_loop(0, _E, _init, 0, unroll=4)

        def _top8(_it, _):
            def _scan(e2, best):
                return jnp.maximum(best, m4_v[e2])
            best = lax.fori_loop(0, _E, _scan, neg, unroll=8)

            def _mark(e2, done):
                v = m4_v[e2]
                pick = jnp.logical_and(v == best, done == 0.0)
                m4_v[e2] = jnp.where(pick, neg, v)
                return jnp.where(pick, one, done)
            lax.fori_loop(0, _E, _mark, jnp.zeros((_L,), jnp.float32),
                          unroll=8)
            return 0
        lax.fori_loop(0, _K, _top8, 0)

        def _wsum(e2, denom):
            w = jnp.where(
                jnp.logical_and(gact_v[e2 // _GS] > 0, m4_v[e2] == neg),
                scores_v[e2], 0.0)
            out_v[e2, 0] = w
            return denom + w
        denom = lax.fori_loop(0, _E, _wsum, jnp.zeros((_L,), jnp.float32),
                              unroll=4)

        scale = _SCALE / (denom + 1e-20)

        def _norm(e2, _):
            out_v[e2, 0] = out_v[e2, 0] * scale
            return 0
        lax.fori_loop(0, _E, _norm, 0, unroll=4)

        pltpu.sync_copy(out_v, wd_hbm.at[:, wid])


_sc_gate = functools.partial(
    pl.kernel,
    out_type=jax.ShapeDtypeStruct((_E, _NCHUNK, 1, _L), jnp.float32),
    mesh=plsc.VectorSubcoreMesh(core_axis_name="c", subcore_axis_name="s"),
    scratch_types=[
        pltpu.VMEM((_E, _L), jnp.float32),
        pltpu.VMEM((_E, _L), jnp.float32),
        pltpu.VMEM((_E, _L), jnp.float32),
        pltpu.VMEM((_E, 1, _L), jnp.float32),
        pltpu.VMEM((_NG, _L), jnp.float32),
        pltpu.VMEM((_NG, _L), jnp.float32),
    ],
)(_sc_gate_body)


_EA = 8


def _mlp(x, wg, wu, wd):
    h = jnp.dot(x, wg, preferred_element_type=jnp.float32)
    u = jnp.dot(x, wu, preferred_element_type=jnp.float32)
    act = (h * jax.nn.sigmoid(h)) * u
    return jnp.dot(act, wd, preferred_element_type=jnp.float32)


def _moe_a_kernel(x_ref, wg_ref, wu_ref, wd_w_ref, y_ref):
    x = x_ref[...]
    for j in range(_EPB):
        y_ref[j] = _mlp(x, wg_ref[j], wu_ref[j], wd_w_ref[j]).astype(
            jnp.bfloat16)


def _moe_b_kernel(x_ref, wdT_ref, y8_ref, wg_ref, wu_ref, wd_w_ref, out_ref,
                  wd_scr):
    e = pl.program_id(0)
    lane = lax.broadcasted_iota(jnp.int32, (_T, _E), 1)

    def _wcol(idx):
        return jnp.sum(jnp.where(lane == idx, wd_scr[...], 0.0),
                       axis=1, keepdims=True)

    @pl.when(e == 0)
    def _():
        ii = lax.broadcasted_iota(jnp.int32, (_E, _E), 0)
        jj = lax.broadcasted_iota(jnp.int32, (_E, _E), 1)
        eye = (ii == jj).astype(jnp.float32)
        wd_scr[...] = lax.dot_general(wdT_ref[...], eye,
                                      (((0,), (0,)), ((), ())),
                                      preferred_element_type=jnp.float32)
        out_ref[...] = jnp.zeros((_T, _D), jnp.float32)

    @pl.when(e < _EA // _EPB)
    def _():
        contrib_a = jnp.zeros((_T, _D), jnp.float32)
        for j in range(_EPB):
            contrib_a = contrib_a + (
                y8_ref[j].astype(jnp.float32) * _wcol(_EPB * e + j))
        out_ref[...] += contrib_a

    x = x_ref[...]
    contrib = jnp.zeros((_T, _D), jnp.float32)
    for j in range(_EPB):
        y = _mlp(x, wg_ref[j], wu_ref[j], wd_w_ref[j])
        contrib = contrib + y * _wcol(_EA + _EPB * e + j)
    out_ref[...] += contrib


def kernel(tt_input, gate_w, gate_bias, w_gate, w_up, w_down):
    gb_col = gate_bias.reshape(_E, 1)
    scores_c, s4c_c = pl.pallas_call(
        _logits_kernel,
        in_specs=[
            pl.BlockSpec((_T, _D), lambda: (0, 0)),
            pl.BlockSpec((_E, _D), lambda: (0, 0)),
            pl.BlockSpec((_E, 1), lambda: (0, 0)),
        ],
        out_specs=[
            pl.BlockSpec((_NCHUNK, _E, _L), lambda: (0, 0, 0)),
            pl.BlockSpec((_NCHUNK, _E, _L), lambda: (0, 0, 0)),
        ],
        out_shape=[
            jax.ShapeDtypeStruct((_NCHUNK, _E, _L), jnp.float32),
            jax.ShapeDtypeStruct((_NCHUNK, _E, _L), jnp.float32),
        ],
    )(tt_input, gate_w, gb_col)

    wdT = _sc_gate(scores_c, s4c_c).reshape(_E, _T)

    y8 = pl.pallas_call(
        _moe_a_kernel,
        grid=(_EA // _EPB,),
        in_specs=[
            pl.BlockSpec((_T, _D), lambda e: (0, 0)),
            pl.BlockSpec((_EPB, _D, _F), lambda e: (e, 0, 0)),
            pl.BlockSpec((_EPB, _D, _F), lambda e: (e, 0, 0)),
            pl.BlockSpec((_EPB, _F, _D), lambda e: (e, 0, 0)),
        ],
        out_specs=pl.BlockSpec((_EPB, _T, _D), lambda e: (e, 0, 0)),
        out_shape=jax.ShapeDtypeStruct((_EA, _T, _D), jnp.bfloat16),
        compiler_params=pltpu.CompilerParams(
            dimension_semantics=("arbitrary",),
        ),
    )(tt_input, w_gate, w_up, w_down)

    nb = (_E - _EA) // _EPB
    return pl.pallas_call(
        _moe_b_kernel,
        grid=(nb,),
        in_specs=[
            pl.BlockSpec((_T, _D), lambda e: (0, 0)),
            pl.BlockSpec((_E, _T), lambda e: (0, 0)),
            pl.BlockSpec((_EPB, _T, _D),
                         lambda e: (jnp.minimum(e, _EA // _EPB - 1), 0, 0)),
            pl.BlockSpec((_EPB, _D, _F), lambda e: (e + _EA // _EPB, 0, 0)),
            pl.BlockSpec((_EPB, _D, _F), lambda e: (e + _EA // _EPB, 0, 0)),
            pl.BlockSpec((_EPB, _F, _D), lambda e: (e + _EA // _EPB, 0, 0)),
        ],
        out_specs=pl.BlockSpec((_T, _D), lambda e: (0, 0)),
        out_shape=jax.ShapeDtypeStruct((_T, _D), jnp.float32),
        scratch_shapes=[pltpu.VMEM((_T, _E), jnp.float32)],
        compiler_params=pltpu.CompilerParams(
            dimension_semantics=("arbitrary",),
        ),
    )(tt_input, wdT, y8, w_gate, w_up, w_down)


# speedup vs baseline: 1.0838x; 1.0112x over previous
import functools

import jax
import jax.numpy as jnp
from jax import lax
from jax.experimental import pallas as pl
from jax.experimental.pallas import tpu as pltpu
from jax.experimental.pallas import tpu_sc as plsc

_E = 64
_K = 8
_NG = 8
_KG = 4
_GS = _E // _NG
_D = 1024
_F = 512
_T = 128
_SCALE = 2.5
_EPB = 2
_L = 16
_NCHUNK = _T // _L


def _logits_kernel(x_ref, gw_ref, gb_ref, scores_ref, s4c_ref):
    lt = lax.dot_general(gw_ref[...], x_ref[...], (((1,), (1,)), ((), ())),
                         preferred_element_type=jnp.float32)
    sc = jax.nn.sigmoid(lt)
    s4c = sc + gb_ref[...]
    for c in range(_NCHUNK):
        scores_ref[c] = sc[:, c * _L:(c + 1) * _L]
        s4c_ref[c] = s4c[:, c * _L:(c + 1) * _L]


def _sc_gate_body(scores_hbm, s4c_hbm, wd_hbm, scores_v, s4_v, m4_v, out_v,
                  gs_v, gact_v):
    wid = lax.axis_index("s") * 2 + lax.axis_index("c")

    @pl.when(wid < _NCHUNK)
    def _():
        pltpu.sync_copy(scores_hbm.at[wid], scores_v)
        pltpu.sync_copy(s4c_hbm.at[wid], s4_v)

        neg = jnp.full((_L,), -jnp.inf, jnp.float32)
        one = jnp.full((_L,), 1.0, jnp.float32)

        def _grp(g, _):
            def _run(j, c):
                m1, m2 = c
                v = s4_v[g * _GS + j]
                gt = v > m1
                return (jnp.maximum(m1, v),
                        jnp.where(gt, m1, jnp.maximum(m2, v)))
            m1, m2 = lax.fori_loop(0, _GS, _run, (neg, neg))
            gs_v[g] = m1 + m2
            gact_v[g] = jnp.zeros((_L,), jnp.float32)
            return 0
        lax.fori_loop(0, _NG, _grp, 0)

        def _g4(_it, _):
            def _gscan(g, best):
                v = jnp.where(gact_v[g] > 0, neg, gs_v[g])
                return jnp.maximum(best, v)
            best = lax.fori_loop(0, _NG, _gscan, neg)

            def _gmark(g, done):
                v = jnp.where(gact_v[g] > 0, neg, gs_v[g])
                pick = jnp.logical_and(v == best, done == 0.0)
                gact_v[g] = jnp.where(pick, one, gact_v[g])
                return jnp.where(pick, one, done)
            lax.fori_loop(0, _NG, _gmark, jnp.zeros((_L,), jnp.float32))
            return 0
        lax.fori_loop(0, _KG, _g4, 0)

        def _init(e2, _):
            m4_v[e2] = jnp.where(gact_v[e2 // _GS] > 0, s4_v[e2], neg)
            return 0
        lax.fori_loop(0, _E, _init, 0, unroll=4)

        def _top8(_it, _):
            def _scan(e2, best):
                return jnp.maximum(best, m4_v[e2])
            best = lax.fori_loop(0, _E, _scan, neg, unroll=8)

            def _mark(e2, done):
                v = m4_v[e2]
                pick = jnp.logical_and(v == best, done == 0.0)
                m4_v[e2] = jnp.where(pick, neg, v)
                return jnp.where(pick, one, done)
            lax.fori_loop(0, _E, _mark, jnp.zeros((_L,), jnp.float32),
                          unroll=8)
            return 0
        lax.fori_loop(0, _K, _top8, 0)

        def _wsum(e2, denom):
            w = jnp.where(
                jnp.logical_and(gact_v[e2 // _GS] > 0, m4_v[e2] == neg),
                scores_v[e2], 0.0)
            out_v[e2, 0] = w
            return denom + w
        denom = lax.fori_loop(0, _E, _wsum, jnp.zeros((_L,), jnp.float32),
                              unroll=4)

        scale = _SCALE / (denom + 1e-20)

        def _norm(e2, _):
            out_v[e2, 0] = out_v[e2, 0] * scale
            return 0
        lax.fori_loop(0, _E, _norm, 0, unroll=4)

        pltpu.sync_copy(out_v, wd_hbm.at[:, wid])


_sc_gate = functools.partial(
    pl.kernel,
    out_type=jax.ShapeDtypeStruct((_E, _NCHUNK, 1, _L), jnp.float32),
    mesh=plsc.VectorSubcoreMesh(core_axis_name="c", subcore_axis_name="s"),
    scratch_types=[
        pltpu.VMEM((_E, _L), jnp.float32),
        pltpu.VMEM((_E, _L), jnp.float32),
        pltpu.VMEM((_E, _L), jnp.float32),
        pltpu.VMEM((_E, 1, _L), jnp.float32),
        pltpu.VMEM((_NG, _L), jnp.float32),
        pltpu.VMEM((_NG, _L), jnp.float32),
    ],
)(_sc_gate_body)


_EA = 4


def _mlp(x, wg, wu, wd):
    h = jnp.dot(x, wg, preferred_element_type=jnp.float32)
    u = jnp.dot(x, wu, preferred_element_type=jnp.float32)
    act = (h * jax.nn.sigmoid(h)) * u
    return jnp.dot(act, wd, preferred_element_type=jnp.float32)


def _moe_a_kernel(x_ref, wg_ref, wu_ref, wd_w_ref, y_ref):
    x = x_ref[...]
    for j in range(_EPB):
        y_ref[j] = _mlp(x, wg_ref[j], wu_ref[j], wd_w_ref[j]).astype(
            jnp.bfloat16)


def _moe_b_kernel(x_ref, wdT_ref, y8_ref, wg_ref, wu_ref, wd_w_ref, out_ref,
                  wd_scr):
    e = pl.program_id(0)
    lane = lax.broadcasted_iota(jnp.int32, (_T, _E), 1)

    def _wcol(idx):
        return jnp.sum(jnp.where(lane == idx, wd_scr[...], 0.0),
                       axis=1, keepdims=True)

    @pl.when(e == 0)
    def _():
        ii = lax.broadcasted_iota(jnp.int32, (_E, _E), 0)
        jj = lax.broadcasted_iota(jnp.int32, (_E, _E), 1)
        eye = (ii == jj).astype(jnp.float32)
        wd_scr[...] = lax.dot_general(wdT_ref[...], eye,
                                      (((0,), (0,)), ((), ())),
                                      preferred_element_type=jnp.float32)
        out_ref[...] = jnp.zeros((_T, _D), jnp.float32)

    @pl.when(e < _EA // _EPB)
    def _():
        contrib_a = jnp.zeros((_T, _D), jnp.float32)
        for j in range(_EPB):
            contrib_a = contrib_a + (
                y8_ref[j].astype(jnp.float32) * _wcol(_EPB * e + j))
        out_ref[...] += contrib_a

    x = x_ref[...]
    contrib = jnp.zeros((_T, _D), jnp.float32)
    for j in range(_EPB):
        y = _mlp(x, wg_ref[j], wu_ref[j], wd_w_ref[j])
        contrib = contrib + y * _wcol(_EA + _EPB * e + j)
    out_ref[...] += contrib


def kernel(tt_input, gate_w, gate_bias, w_gate, w_up, w_down):
    gb_col = gate_bias.reshape(_E, 1)
    scores_c, s4c_c = pl.pallas_call(
        _logits_kernel,
        in_specs=[
            pl.BlockSpec((_T, _D), lambda: (0, 0)),
            pl.BlockSpec((_E, _D), lambda: (0, 0)),
            pl.BlockSpec((_E, 1), lambda: (0, 0)),
        ],
        out_specs=[
            pl.BlockSpec((_NCHUNK, _E, _L), lambda: (0, 0, 0)),
            pl.BlockSpec((_NCHUNK, _E, _L), lambda: (0, 0, 0)),
        ],
        out_shape=[
            jax.ShapeDtypeStruct((_NCHUNK, _E, _L), jnp.float32),
            jax.ShapeDtypeStruct((_NCHUNK, _E, _L), jnp.float32),
        ],
    )(tt_input, gate_w, gb_col)

    wdT = _sc_gate(scores_c, s4c_c).reshape(_E, _T)

    y8 = pl.pallas_call(
        _moe_a_kernel,
        grid=(_EA // _EPB,),
        in_specs=[
            pl.BlockSpec((_T, _D), lambda e: (0, 0)),
            pl.BlockSpec((_EPB, _D, _F), lambda e: (e, 0, 0)),
            pl.BlockSpec((_EPB, _D, _F), lambda e: (e, 0, 0)),
            pl.BlockSpec((_EPB, _F, _D), lambda e: (e, 0, 0)),
        ],
        out_specs=pl.BlockSpec((_EPB, _T, _D), lambda e: (e, 0, 0)),
        out_shape=jax.ShapeDtypeStruct((_EA, _T, _D), jnp.bfloat16),
        compiler_params=pltpu.CompilerParams(
            dimension_semantics=("arbitrary",),
        ),
    )(tt_input, w_gate, w_up, w_down)

    nb = (_E - _EA) // _EPB
    return pl.pallas_call(
        _moe_b_kernel,
        grid=(nb,),
        in_specs=[
            pl.BlockSpec((_T, _D), lambda e: (0, 0)),
            pl.BlockSpec((_E, _T), lambda e: (0, 0)),
            pl.BlockSpec((_EPB, _T, _D),
                         lambda e: (jnp.minimum(e, _EA // _EPB - 1), 0, 0)),
            pl.BlockSpec((_EPB, _D, _F), lambda e: (e + _EA // _EPB, 0, 0)),
            pl.BlockSpec((_EPB, _D, _F), lambda e: (e + _EA // _EPB, 0, 0)),
            pl.BlockSpec((_EPB, _F, _D), lambda e: (e + _EA // _EPB, 0, 0)),
        ],
        out_specs=pl.BlockSpec((_T, _D), lambda e: (0, 0)),
        out_shape=jax.ShapeDtypeStruct((_T, _D), jnp.float32),
        scratch_shapes=[pltpu.VMEM((_T, _E), jnp.float32)],
        compiler_params=pltpu.CompilerParams(
            dimension_semantics=("arbitrary",),
        ),
    )(tt_input, wdT, y8, w_gate, w_up, w_down)
